# baseline (speedup 1.0000x reference)
import functools
import math

import jax
import jax.numpy as jnp
from jax import lax
from jax.experimental import pallas as pl
from jax.experimental.pallas import tpu as pltpu
from jax.experimental.pallas import tpu_sc as plsc

B, N, T, D = 16, 1024, 64, 768
NC, NS, L = 2, 16, 16
NW = NC * NS
TOTAL = B * N * N
ROWS = B * N
ROWS_W = ROWS // NW
CHUNK = 128
ZCH = 32768
SPAN_W = TOTAL // NW


def _rms(x, eps=1e-6):
    var = jnp.mean(x * x, axis=-1, keepdims=True)
    return x * lax.rsqrt(var + eps)


def _tc_core(p_ref, t_ref, idx_ref):
    p = p_ref[0]
    t = t_ref[0]
    scale = 1.0 / math.sqrt(D)

    pn = _rms(p)
    tn = _rms(t)
    logits = lax.dot_general(
        pn, tn, (((1,), (1,)), ((), ())),
        preferred_element_type=jnp.float32) * scale
    attn = jax.nn.softmax(logits, axis=-1)
    q = lax.dot_general(
        attn, tn, (((1,), (0,)), ((), ())),
        preferred_element_type=jnp.float32)
    qn = _rms(q)
    score = lax.dot_general(
        qn, pn, (((1,), (1,)), ((), ())),
        preferred_element_type=jnp.float32) * scale
    idx_ref[0] = jnp.argmax(score, axis=-1)[None, :]


_mesh = plsc.VectorSubcoreMesh(core_axis_name="c", subcore_axis_name="s")


@functools.partial(
    pl.kernel, mesh=_mesh,
    out_type=jax.ShapeDtypeStruct((TOTAL,), jnp.float32),
    scratch_types=[
        pltpu.VMEM((ZCH,), jnp.float32),
        pltpu.SemaphoreType.DMA,
    ],
)
def _sc_memset(out_hbm, zbuf, sem):
    wid = lax.axis_index("s") * NC + lax.axis_index("c")
    base = wid * SPAN_W

    def zfill(i, carry):
        zbuf[pl.ds(i * L, L)] = jnp.zeros((L,), jnp.float32)
        return carry
    lax.fori_loop(0, ZCH // L, zfill, 0)

    copies = [
        pltpu.make_async_copy(
            zbuf, out_hbm.at[pl.ds(base + k * ZCH, ZCH)], sem)
        for k in range(SPAN_W // ZCH)
    ]
    for cp in copies:
        cp.start()
    for cp in copies:
        cp.wait()


@functools.partial(
    pl.kernel, mesh=_mesh,
    out_type=(),
    scratch_types=[
        pltpu.VMEM((ROWS_W,), jnp.int32),
        pltpu.VMEM((CHUNK,), jnp.int32),
        pltpu.VMEM((CHUNK,), jnp.float32),
        pltpu.SemaphoreType.DMA,
    ],
)
def _sc_scatter(idx_hbm, out_ref, idxv, obuf, ones, sem):
    wid = lax.axis_index("s") * NC + lax.axis_index("c")
    base = wid * ROWS_W
    pltpu.sync_copy(idx_hbm.at[pl.ds(base, ROWS_W)], idxv)
    for i in range(CHUNK // L):
        ones[pl.ds(i * L, L)] = jnp.full((L,), 1.0, jnp.float32)
    lane = lax.iota(jnp.int32, L)
    for c in range(ROWS_W // CHUNK):
        for j in range(CHUNK // L):
            r = c * CHUNK + j * L
            rows = base + r + lane
            obuf[pl.ds(j * L, L)] = rows * N + idxv[pl.ds(r, L)]
        pltpu.async_copy(ones, out_ref.at[obuf], sem).wait()


def kernel(patches, task_tokens):
    idx3 = pl.pallas_call(
        _tc_core,
        grid=(B,),
        in_specs=[
            pl.BlockSpec((1, N, D), lambda i: (i, 0, 0)),
            pl.BlockSpec((1, T, D), lambda i: (i, 0, 0)),
        ],
        out_specs=pl.BlockSpec((1, 1, N), lambda i: (i, 0, 0)),
        out_shape=jax.ShapeDtypeStruct((B, 1, N), jnp.int32),
    )(patches, task_tokens)
    idx_flat = idx3.reshape(ROWS)

    zeros = _sc_memset()
    ref = jax.new_ref(zeros)
    _sc_scatter(idx_flat, ref)
    return ref[...].reshape(B, N, N)


# speedup vs baseline: 2.2661x; 2.2661x over previous
import math

import jax
import jax.numpy as jnp
from jax import lax
from jax.experimental import pallas as pl


def _rms(x, eps=1e-6):
    var = jnp.mean(x * x, axis=-1, keepdims=True)
    return x * lax.rsqrt(var + eps)


def _core(p_ref, t_ref, o_ref):
    p = p_ref[0]
    t = t_ref[0]
    n, d = p.shape
    scale = 1.0 / math.sqrt(d)

    pn = _rms(p)
    tn = _rms(t)

    h = n // 2
    for k in range(2):
        rows = pn[k * h:(k + 1) * h]
        logits = lax.dot_general(
            rows, tn, (((1,), (1,)), ((), ())),
            preferred_element_type=jnp.float32) * scale
        attn = jax.nn.softmax(logits, axis=-1)
        q = lax.dot_general(
            attn, tn, (((1,), (0,)), ((), ())),
            preferred_element_type=jnp.float32)
        qn = _rms(q)
        score = lax.dot_general(
            qn, pn, (((1,), (1,)), ((), ())),
            preferred_element_type=jnp.float32) * scale
        idx = jnp.argmax(score, axis=-1)
        cols = lax.broadcasted_iota(jnp.int32, score.shape, 1)
        o_ref[0, k * h:(k + 1) * h] = jnp.where(
            cols == idx[:, None], 1.0, 0.0).astype(jnp.float32)


def kernel(patches, task_tokens):
    b, n, d = patches.shape
    t = task_tokens.shape[1]
    return pl.pallas_call(
        _core,
        grid=(b,),
        in_specs=[
            pl.BlockSpec((1, n, d), lambda i: (i, 0, 0)),
            pl.BlockSpec((1, t, d), lambda i: (i, 0, 0)),
        ],
        out_specs=pl.BlockSpec((1, n, n), lambda i: (i, 0, 0)),
        out_shape=jax.ShapeDtypeStruct((b, n, n), jnp.float32),
    )(patches, task_tokens)


# speedup vs baseline: 2.4655x; 1.0880x over previous
import math

import jax
import jax.numpy as jnp
from jax import lax
from jax.experimental import pallas as pl


def _rms(x, eps=1e-6):
    var = jnp.mean(x * x, axis=-1, keepdims=True)
    return x * lax.rsqrt(var + eps)


def _core(p_ref, t_ref, o_ref):
    d = p_ref.shape[-1]
    scale = 1.0 / math.sqrt(d)
    for k in range(p_ref.shape[0]):
        p = p_ref[k]
        t = t_ref[k]
        pn = _rms(p)
        tn = _rms(t)
        logits = lax.dot_general(
            pn, tn, (((1,), (1,)), ((), ())),
            preferred_element_type=jnp.float32) * scale
        attn = jax.nn.softmax(logits, axis=-1)
        q = lax.dot_general(
            attn, tn, (((1,), (0,)), ((), ())),
            preferred_element_type=jnp.float32)
        qn = _rms(q)
        score = lax.dot_general(
            qn, pn, (((1,), (1,)), ((), ())),
            preferred_element_type=jnp.float32) * scale
        idx = jnp.argmax(score, axis=-1)
        cols = lax.broadcasted_iota(jnp.int32, score.shape, 1)
        o_ref[k] = jnp.where(cols == idx[:, None], 1.0, 0.0).astype(jnp.float32)


def kernel(patches, task_tokens):
    b, n, d = patches.shape
    t = task_tokens.shape[1]
    bb = 2
    return pl.pallas_call(
        _core,
        grid=(b // bb,),
        in_specs=[
            pl.BlockSpec((bb, n, d), lambda i: (i, 0, 0)),
            pl.BlockSpec((bb, t, d), lambda i: (i, 0, 0)),
        ],
        out_specs=pl.BlockSpec((bb, n, n), lambda i: (i, 0, 0)),
        out_shape=jax.ShapeDtypeStruct((b, n, n), jnp.float32),
    )(patches, task_tokens)


# speedup vs baseline: 2.6401x; 1.0708x over previous
import math

import jax
import jax.numpy as jnp
from jax import lax
from jax.experimental import pallas as pl


def _rms(x, eps=1e-6):
    var = jnp.mean(x * x, axis=-1, keepdims=True)
    return x * lax.rsqrt(var + eps)


def _core(p_ref, t_ref, o_ref):
    p = p_ref[0]
    t = t_ref[0]
    d = p.shape[-1]
    scale = 1.0 / math.sqrt(d)

    pn = _rms(p)
    tn = _rms(t)
    logits = lax.dot_general(
        pn, tn, (((1,), (1,)), ((), ())),
        preferred_element_type=jnp.float32) * scale
    attn = jax.nn.softmax(logits, axis=-1)
    q = lax.dot_general(
        attn, tn, (((1,), (0,)), ((), ())),
        preferred_element_type=jnp.float32)
    qn = _rms(q)
    score = lax.dot_general(
        qn, pn, (((1,), (1,)), ((), ())),
        preferred_element_type=jnp.float32) * scale
    m = jnp.max(score, axis=-1, keepdims=True)
    o_ref[0] = jnp.where(score == m, 1.0, 0.0).astype(jnp.float32)


def kernel(patches, task_tokens):
    b, n, d = patches.shape
    t = task_tokens.shape[1]
    return pl.pallas_call(
        _core,
        grid=(b,),
        in_specs=[
            pl.BlockSpec((1, n, d), lambda i: (i, 0, 0)),
            pl.BlockSpec((1, t, d), lambda i: (i, 0, 0)),
        ],
        out_specs=pl.BlockSpec((1, n, n), lambda i: (i, 0, 0)),
        out_shape=jax.ShapeDtypeStruct((b, n, n), jnp.float32),
    )(patches, task_tokens)
